# baseline (speedup 1.0000x reference)
import functools

import jax
import jax.numpy as jnp
import numpy as np
from jax import lax
from jax.experimental import pallas as pl
from jax.experimental.pallas import tpu as pltpu
from jax.experimental.pallas import tpu_sc as plsc

B, L, D, V = 4096, 50, 256, 1000
VW = 256
VP = 4 * VW
BLK = 1024

NC, NS = 2, 16
NW = NC * NS
RPW = B // NW
G = 16
NG = RPW // G

_j = np.arange(VP)
STACK_V = 4 * (_j % VW) + _j // VW
STACK_VALID = STACK_V < V
STACK_IDX = np.where(STACK_VALID, STACK_V, 0)


def _sc_counts_body(ids_hbm, out_hbm, ids_v, tile_a, tile_b, sem_a, sem_b):
    wid = lax.axis_index("s") * NC + lax.axis_index("c")
    pltpu.sync_copy(ids_hbm.at[pl.ds(wid * RPW, RPW), :], ids_v)

    lane = lax.broadcasted_iota(jnp.int32, (G,), 0)
    zeros = jnp.zeros((G,), jnp.int32)

    pending = [None, None]
    for g in range(NG):
        tile = tile_a if g % 2 == 0 else tile_b
        sem = sem_a if g % 2 == 0 else sem_b
        if pending[g % 2] is not None:
            pending[g % 2].wait()

        def zero_tile(k, _):
            for j in range(G):
                tile[j, pl.ds(k * 16, 16)] = zeros
            return 0
        lax.fori_loop(0, VW // 16, zero_tile, 0)

        rows = lane + g * G
        ones = jnp.full((G,), 1, jnp.int32)

        def scat1(l):
            ids_vec = plsc.load_gather(
                ids_v, [rows, jnp.broadcast_to(l, (G,)).astype(jnp.int32)])
            val = jnp.left_shift(ones, (ids_vec & 3) * 8)
            plsc.addupdate_scatter(tile, [lane, ids_vec >> 2], val)

        def scat(i, _):
            for u in range(5):
                scat1(i * 5 + u)
            return 0
        lax.fori_loop(0, L // 5, scat, 0)

        cp = pltpu.make_async_copy(
            tile, out_hbm.at[pl.ds(wid * RPW + g * G, G), :], sem)
        cp.start()
        pending[g % 2] = cp
    pending[0].wait()
    pending[1].wait()


def _sc_counts(ids_grouped):
    mesh = plsc.VectorSubcoreMesh(core_axis_name="c", subcore_axis_name="s")
    run = functools.partial(
        pl.kernel,
        mesh=mesh,
        compiler_params=pltpu.CompilerParams(needs_layout_passes=False),
        out_type=jax.ShapeDtypeStruct((B, VW), jnp.int32),
        scratch_types=[
            pltpu.VMEM((RPW, L), jnp.int32),
            pltpu.VMEM((G, VW), jnp.int32),
            pltpu.VMEM((G, VW), jnp.int32),
            pltpu.SemaphoreType.DMA,
            pltpu.SemaphoreType.DMA,
        ],
    )(_sc_counts_body)
    return run(ids_grouped)


def _tc_body(cnt_ref, pos_ref, table_ref, aux_ref, W1_ref, W2_ref, out_ref):
    w = cnt_ref[...]
    counts = jnp.concatenate(
        [((w >> (8 * q)) & 0xFF).astype(jnp.float32) for q in range(4)],
        axis=1)
    sym_mean = jnp.dot(counts, table_ref[...],
                       preferred_element_type=jnp.float32) * (1.0 / L)

    pos = pos_ref[...]
    par = lax.broadcasted_iota(jnp.int32, (1, 2 * L), 1) % 2
    px = jnp.sum(jnp.where(par == 0, pos, 0.0), axis=1, keepdims=True) * (1.0 / L)
    py = jnp.sum(jnp.where(par == 1, pos, 0.0), axis=1, keepdims=True) * (1.0 / L)
    aux = aux_ref[...]
    pos_pool = px * aux[0:1, :] + py * aux[1:2, :] + aux[2:3, :]

    pre = (jnp.dot(sym_mean, W1_ref[0:D, :], preferred_element_type=jnp.float32)
           + jnp.dot(pos_pool, W1_ref[D:2 * D, :], preferred_element_type=jnp.float32)
           + aux[3:4, :])
    h = 0.5 * pre * (1.0 + jax.lax.erf(pre * (2.0 ** -0.5)))
    out_ref[...] = jnp.dot(h, W2_ref[...],
                           preferred_element_type=jnp.float32) + aux[4:5, :]


def kernel(symbol_ids, positions, embed_table, pos_W, pos_b, W1, b1, W2, b2):
    counts_w = _sc_counts(symbol_ids)

    table_stack = jnp.where(
        jnp.asarray(STACK_VALID[:, None]),
        embed_table[jnp.asarray(STACK_IDX), :],
        0.0,
    )

    pos_flat = positions.reshape(B, 2 * L)
    aux = jnp.concatenate([
        pos_W,
        pos_b[None, :], b1[None, :], b2[None, :],
        jnp.zeros((3, D), jnp.float32),
    ], axis=0)

    grid = (B // BLK,)
    return pl.pallas_call(
        _tc_body,
        grid=grid,
        in_specs=[
            pl.BlockSpec((BLK, VW), lambda i: (i, 0)),
            pl.BlockSpec((BLK, 2 * L), lambda i: (i, 0)),
            pl.BlockSpec((VP, D), lambda i: (0, 0)),
            pl.BlockSpec((8, D), lambda i: (0, 0)),
            pl.BlockSpec((2 * D, D), lambda i: (0, 0)),
            pl.BlockSpec((D, D), lambda i: (0, 0)),
        ],
        out_specs=pl.BlockSpec((BLK, D), lambda i: (i, 0)),
        out_shape=jax.ShapeDtypeStruct((B, D), jnp.float32),
    )(counts_w, pos_flat, table_stack, aux, W1, W2)


# speedup vs baseline: 1.0841x; 1.0841x over previous
import functools

import jax
import jax.numpy as jnp
import numpy as np
from jax import lax
from jax.experimental import pallas as pl
from jax.experimental.pallas import tpu as pltpu
from jax.experimental.pallas import tpu_sc as plsc

B, L, D, V = 4096, 50, 256, 1000
VW = 256
VP = 4 * VW
BLK = 1024

NC, NS = 2, 16
NW = NC * NS
RPW = B // NW
G = 16
NG = RPW // G

_j = np.arange(VP)
STACK_V = 4 * (_j % VW) + _j // VW
STACK_VALID = STACK_V < V
STACK_IDX = np.where(STACK_VALID, STACK_V, 0)


def _sc_counts_body(ids_hbm, out_hbm, ids_v, tile_a, tile_b, sem_a, sem_b):
    wid = lax.axis_index("s") * NC + lax.axis_index("c")
    pltpu.sync_copy(ids_hbm.at[wid], ids_v)

    lane = lax.broadcasted_iota(jnp.int32, (G,), 0)
    zeros = jnp.zeros((G,), jnp.int32)

    pending = [None, None]
    for g in range(NG):
        tile = tile_a if g % 2 == 0 else tile_b
        sem = sem_a if g % 2 == 0 else sem_b
        if pending[g % 2] is not None:
            pending[g % 2].wait()

        def zero_tile(k, _):
            for j in range(G):
                tile[j, pl.ds(k * 16, 16)] = zeros
            return 0
        lax.fori_loop(0, VW // 16, zero_tile, 0)

        lane_l = lane * L
        ones = jnp.full((G,), 1, jnp.int32)

        def scat1(l):
            ids_vec = plsc.load_gather(ids_v, [lane_l + (g * G * L + l)])
            val = jnp.left_shift(ones, (ids_vec & 3) * 8)
            plsc.addupdate_scatter(tile, [lane, ids_vec >> 2], val)

        def scat(i, _):
            for u in range(5):
                scat1(i * 5 + u)
            return 0
        lax.fori_loop(0, L // 5, scat, 0)

        cp = pltpu.make_async_copy(
            tile, out_hbm.at[pl.ds(wid * RPW + g * G, G), :], sem)
        cp.start()
        pending[g % 2] = cp
    pending[0].wait()
    pending[1].wait()


def _sc_counts(ids_grouped):
    mesh = plsc.VectorSubcoreMesh(core_axis_name="c", subcore_axis_name="s")
    run = functools.partial(
        pl.kernel,
        mesh=mesh,
        compiler_params=pltpu.CompilerParams(needs_layout_passes=False),
        out_type=jax.ShapeDtypeStruct((B, VW), jnp.int32),
        scratch_types=[
            pltpu.VMEM((RPW * L,), jnp.int32),
            pltpu.VMEM((G, VW), jnp.int32),
            pltpu.VMEM((G, VW), jnp.int32),
            pltpu.SemaphoreType.DMA,
            pltpu.SemaphoreType.DMA,
        ],
    )(_sc_counts_body)
    return run(ids_grouped)


def _tc_body(cnt_ref, pos_ref, table_ref, aux_ref, W1_ref, W2_ref, out_ref):
    w = cnt_ref[...]
    counts = jnp.concatenate(
        [((w >> (8 * q)) & 0xFF).astype(jnp.float32) for q in range(4)],
        axis=1)
    sym_mean = jnp.dot(counts, table_ref[...],
                       preferred_element_type=jnp.float32) * (1.0 / L)

    pos = pos_ref[...]
    par = lax.broadcasted_iota(jnp.int32, (1, 2 * L), 1) % 2
    px = jnp.sum(jnp.where(par == 0, pos, 0.0), axis=1, keepdims=True) * (1.0 / L)
    py = jnp.sum(jnp.where(par == 1, pos, 0.0), axis=1, keepdims=True) * (1.0 / L)
    aux = aux_ref[...]
    pos_pool = px * aux[0:1, :] + py * aux[1:2, :] + aux[2:3, :]

    pre = (jnp.dot(sym_mean, W1_ref[0:D, :], preferred_element_type=jnp.float32)
           + jnp.dot(pos_pool, W1_ref[D:2 * D, :], preferred_element_type=jnp.float32)
           + aux[3:4, :])
    h = 0.5 * pre * (1.0 + jax.lax.erf(pre * (2.0 ** -0.5)))
    out_ref[...] = jnp.dot(h, W2_ref[...],
                           preferred_element_type=jnp.float32) + aux[4:5, :]


def kernel(symbol_ids, positions, embed_table, pos_W, pos_b, W1, b1, W2, b2):
    ids_grouped = symbol_ids.reshape(NW, RPW * L)
    counts_w = _sc_counts(ids_grouped)

    table_stack = jnp.where(
        jnp.asarray(STACK_VALID[:, None]),
        embed_table[jnp.asarray(STACK_IDX), :],
        0.0,
    )

    pos_flat = positions.reshape(B, 2 * L)
    aux = jnp.concatenate([
        pos_W,
        pos_b[None, :], b1[None, :], b2[None, :],
        jnp.zeros((3, D), jnp.float32),
    ], axis=0)

    grid = (B // BLK,)
    return pl.pallas_call(
        _tc_body,
        grid=grid,
        in_specs=[
            pl.BlockSpec((BLK, VW), lambda i: (i, 0)),
            pl.BlockSpec((BLK, 2 * L), lambda i: (i, 0)),
            pl.BlockSpec((VP, D), lambda i: (0, 0)),
            pl.BlockSpec((8, D), lambda i: (0, 0)),
            pl.BlockSpec((2 * D, D), lambda i: (0, 0)),
            pl.BlockSpec((D, D), lambda i: (0, 0)),
        ],
        out_specs=pl.BlockSpec((BLK, D), lambda i: (i, 0)),
        out_shape=jax.ShapeDtypeStruct((B, D), jnp.float32),
    )(counts_w, pos_flat, table_stack, aux, W1, W2)
